# baseline (device time: 443206 ns/iter reference)
import jax
import jax.numpy as jnp
from jax import lax
from jax.experimental import pallas as pl
from jax.experimental.pallas import tpu as pltpu

T = 8
NSLOT = 4


def kernel(A, B):
    M, K = A.shape
    _, N = B.shape
    TN = N // T

    a = A.astype(jnp.bfloat16)
    b = B.astype(jnp.bfloat16)

    def body(a_ref, b_ref, out_ref, ptheirs_ref, work_ref, recv_buf,
             send_sems, recv_sems, copy_sems):
        my_x = lax.axis_index("x")
        my_y = lax.axis_index("y")
        peer = (1 - my_x, my_y)

        barrier = pltpu.get_barrier_semaphore()
        pl.semaphore_signal(barrier, inc=1, device_id=peer,
                            device_id_type=pl.DeviceIdType.MESH)
        pl.semaphore_wait(barrier, 1)

        def rdma_for(t, slot):
            return pltpu.make_async_remote_copy(
                src_ref=work_ref.at[slot],
                dst_ref=ptheirs_ref.at[t],
                send_sem=send_sems.at[t],
                recv_sem=recv_sems.at[t],
                device_id=peer,
                device_id_type=pl.DeviceIdType.MESH,
            )

        def step(t, carry):
            @pl.when(t < T)
            def _():
                slot = lax.rem(t, NSLOT)

                @pl.when(t >= NSLOT)
                def _():
                    rdma_for(t - NSLOT, slot).wait_send()

                work_ref[slot] = jnp.dot(
                    a_ref[...], b_ref[:, pl.ds(t * TN, TN)],
                    preferred_element_type=jnp.float32,
                ).astype(jnp.bfloat16)
                rdma_for(t, slot).start()

            @pl.when(t >= 2)
            def _():
                u = t - 2
                uslot = lax.rem(u, NSLOT)
                rdma_for(u, uslot).wait_recv()
                load = pltpu.make_async_copy(
                    ptheirs_ref.at[u], recv_buf, copy_sems.at[0])
                load.start()
                load.wait()
                recv_buf[...] = (
                    work_ref[uslot].astype(jnp.float32)
                    + recv_buf[...].astype(jnp.float32)
                ).astype(jnp.bfloat16)
                store = pltpu.make_async_copy(
                    recv_buf, out_ref.at[:, pl.ds(u * TN, TN)],
                    copy_sems.at[1])
                store.start()
                store.wait()

            return carry

        lax.fori_loop(0, T + 2, step, 0)
        for t in range(T - NSLOT, T):
            rdma_for(t, t % NSLOT).wait_send()

    out, _ = pl.pallas_call(
        body,
        out_shape=[
            jax.ShapeDtypeStruct((M, N), jnp.bfloat16),
            jax.ShapeDtypeStruct((T, M, TN), jnp.bfloat16),
        ],
        in_specs=[
            pl.BlockSpec(memory_space=pltpu.MemorySpace.VMEM),
            pl.BlockSpec(memory_space=pltpu.MemorySpace.VMEM),
        ],
        out_specs=[
            pl.BlockSpec(memory_space=pl.ANY),
            pl.BlockSpec(memory_space=pl.ANY),
        ],
        scratch_shapes=[
            pltpu.VMEM((NSLOT, M, TN), jnp.bfloat16),
            pltpu.VMEM((M, TN), jnp.bfloat16),
            pltpu.SemaphoreType.DMA((T,)),
            pltpu.SemaphoreType.DMA((T,)),
            pltpu.SemaphoreType.DMA((2,)),
        ],
        compiler_params=pltpu.CompilerParams(
            collective_id=0,
            vmem_limit_bytes=64 * 1024 * 1024,
        ),
    )(a, b)
    return out


# device time: 431931 ns/iter; 1.0261x vs baseline; 1.0261x over previous
import jax
import jax.numpy as jnp
from jax import lax
from jax.experimental import pallas as pl
from jax.experimental.pallas import tpu as pltpu

T = 16
NSLOT = 4
CA = 8


def kernel(A, B):
    M, K = A.shape
    _, N = B.shape
    TN = N // T
    CK = K // CA

    def body(a_hbm, b_hbm, out_ref, ptheirs_ref,
             a_bf, a_stage, b_stage, b_bf, work_ref, load_buf, sum_buf,
             a_sems, b_sems, send_sems, recv_sems, copy_sems):
        my_x = lax.axis_index("x")
        my_y = lax.axis_index("y")
        peer = (1 - my_x, my_y)

        barrier = pltpu.get_barrier_semaphore()
        pl.semaphore_signal(barrier, inc=1, device_id=peer,
                            device_id_type=pl.DeviceIdType.MESH)
        pl.semaphore_wait(barrier, 1)

        def a_load(c, slot):
            return pltpu.make_async_copy(
                a_hbm.at[:, pl.ds(c * CK, CK)], a_stage.at[slot],
                a_sems.at[slot])

        a_load(0, 0).start()

        def a_step(c, carry):
            slot = lax.rem(c, 2)

            @pl.when(c < CA - 1)
            def _():
                a_load(c + 1, 1 - slot).start()

            a_load(c, slot).wait()
            a_bf[:, pl.ds(c * CK, CK)] = a_stage[slot].astype(jnp.bfloat16)
            return carry

        lax.fori_loop(0, CA, a_step, 0)

        def b_load(t, slot):
            return pltpu.make_async_copy(
                b_hbm.at[:, pl.ds(t * TN, TN)], b_stage.at[slot],
                b_sems.at[slot])

        b_load(0, 0).start()

        def rdma_for(t, slot):
            return pltpu.make_async_remote_copy(
                src_ref=work_ref.at[slot],
                dst_ref=ptheirs_ref.at[t],
                send_sem=send_sems.at[t],
                recv_sem=recv_sems.at[t],
                device_id=peer,
                device_id_type=pl.DeviceIdType.MESH,
            )

        def step(t, carry):
            @pl.when(t < T)
            def _():
                slot = lax.rem(t, NSLOT)
                bslot = lax.rem(t, 2)

                @pl.when(t >= NSLOT)
                def _():
                    rdma_for(t - NSLOT, slot).wait_send()

                @pl.when(t < T - 1)
                def _():
                    b_load(t + 1, 1 - bslot).start()

                b_load(t, bslot).wait()
                b_bf[bslot] = b_stage[bslot].astype(jnp.bfloat16)
                work_ref[slot] = jnp.dot(
                    a_bf[...], b_bf[bslot],
                    preferred_element_type=jnp.float32,
                ).astype(jnp.bfloat16)
                rdma_for(t, slot).start()

            @pl.when(t >= 2)
            def _():
                u = t - 2
                uslot = lax.rem(u, NSLOT)
                rdma_for(u, uslot).wait_recv()
                load = pltpu.make_async_copy(
                    ptheirs_ref.at[u], load_buf, copy_sems.at[0])
                load.start()
                load.wait()
                sum_buf[...] = (
                    work_ref[uslot].astype(jnp.float32)
                    + load_buf[...].astype(jnp.float32)
                )
                store = pltpu.make_async_copy(
                    sum_buf, out_ref.at[:, pl.ds(u * TN, TN)],
                    copy_sems.at[1])
                store.start()
                store.wait()

            return carry

        lax.fori_loop(0, T + 2, step, 0)
        for t in range(T - NSLOT, T):
            rdma_for(t, t % NSLOT).wait_send()

    out, _ = pl.pallas_call(
        body,
        out_shape=[
            jax.ShapeDtypeStruct((M, N), jnp.float32),
            jax.ShapeDtypeStruct((T, M, TN), jnp.bfloat16),
        ],
        in_specs=[
            pl.BlockSpec(memory_space=pl.ANY),
            pl.BlockSpec(memory_space=pl.ANY),
        ],
        out_specs=[
            pl.BlockSpec(memory_space=pl.ANY),
            pl.BlockSpec(memory_space=pl.ANY),
        ],
        scratch_shapes=[
            pltpu.VMEM((M, K), jnp.bfloat16),
            pltpu.VMEM((2, M, CK), jnp.float32),
            pltpu.VMEM((2, K, TN), jnp.float32),
            pltpu.VMEM((2, K, TN), jnp.bfloat16),
            pltpu.VMEM((NSLOT, M, TN), jnp.bfloat16),
            pltpu.VMEM((M, TN), jnp.bfloat16),
            pltpu.VMEM((M, TN), jnp.float32),
            pltpu.SemaphoreType.DMA((2,)),
            pltpu.SemaphoreType.DMA((2,)),
            pltpu.SemaphoreType.DMA((T,)),
            pltpu.SemaphoreType.DMA((T,)),
            pltpu.SemaphoreType.DMA((2,)),
        ],
        compiler_params=pltpu.CompilerParams(
            collective_id=0,
            vmem_limit_bytes=64 * 1024 * 1024,
        ),
    )(A, B)
    return out


# device time: 429450 ns/iter; 1.0320x vs baseline; 1.0058x over previous
import jax
import jax.numpy as jnp
from jax import lax
from jax.experimental import pallas as pl
from jax.experimental.pallas import tpu as pltpu

T = 16
NSLOT = 4
CA = 4


def kernel(A, B):
    M, K = A.shape
    _, N = B.shape
    TN = N // T
    CK = K // CA

    def body(a_hbm, b_hbm, out_ref, ptheirs_ref,
             a_bf, a_stage, b_stage, b_bf, work_ref, load_buf, sum_buf,
             a_sems, b_sems, send_sems, recv_sems, copy_sems):
        my_x = lax.axis_index("x")
        my_y = lax.axis_index("y")
        peer = (1 - my_x, my_y)

        barrier = pltpu.get_barrier_semaphore()
        pl.semaphore_signal(barrier, inc=1, device_id=peer,
                            device_id_type=pl.DeviceIdType.MESH)
        pl.semaphore_wait(barrier, 1)

        def b_load(t, slot):
            return pltpu.make_async_copy(
                b_hbm.at[:, pl.ds(t * TN, TN)], b_stage.at[slot],
                b_sems.at[slot])

        def a_load(c, slot):
            return pltpu.make_async_copy(
                a_hbm.at[:, pl.ds(c * CK, CK)], a_stage.at[slot],
                a_sems.at[slot])

        b_load(0, 0).start()
        a_load(0, 0).start()

        def a_step(c, carry):
            slot = lax.rem(c, 2)

            @pl.when(c < CA - 1)
            def _():
                a_load(c + 1, 1 - slot).start()

            a_load(c, slot).wait()
            a_bf[:, pl.ds(c * CK, CK)] = a_stage[slot].astype(jnp.bfloat16)
            return carry

        lax.fori_loop(0, CA, a_step, 0)

        def rdma_for(t, slot):
            return pltpu.make_async_remote_copy(
                src_ref=work_ref.at[slot],
                dst_ref=ptheirs_ref.at[t],
                send_sem=send_sems.at[t],
                recv_sem=recv_sems.at[t],
                device_id=peer,
                device_id_type=pl.DeviceIdType.MESH,
            )

        def step(t, carry):
            @pl.when(t < T)
            def _():
                slot = lax.rem(t, NSLOT)
                bslot = lax.rem(t, 2)

                @pl.when(t >= NSLOT)
                def _():
                    rdma_for(t - NSLOT, slot).wait_send()

                @pl.when(t < T - 1)
                def _():
                    b_load(t + 1, 1 - bslot).start()

                b_load(t, bslot).wait()
                b_bf[bslot] = b_stage[bslot].astype(jnp.bfloat16)
                work_ref[slot] = jnp.dot(
                    a_bf[...], b_bf[bslot],
                    preferred_element_type=jnp.float32,
                ).astype(jnp.bfloat16)
                rdma_for(t, slot).start()

            @pl.when(t >= 2)
            def _():
                u = t - 2
                uslot = lax.rem(u, NSLOT)
                rdma_for(u, uslot).wait_recv()
                load = pltpu.make_async_copy(
                    ptheirs_ref.at[u], load_buf, copy_sems.at[0])
                load.start()
                load.wait()
                sum_buf[...] = (
                    work_ref[uslot].astype(jnp.float32)
                    + load_buf[...].astype(jnp.float32)
                )
                store = pltpu.make_async_copy(
                    sum_buf, out_ref.at[:, pl.ds(u * TN, TN)],
                    copy_sems.at[1])
                store.start()
                store.wait()

            return carry

        lax.fori_loop(0, T + 2, step, 0)
        for t in range(T - NSLOT, T):
            rdma_for(t, t % NSLOT).wait_send()

    out, _ = pl.pallas_call(
        body,
        out_shape=[
            jax.ShapeDtypeStruct((M, N), jnp.float32),
            jax.ShapeDtypeStruct((T, M, TN), jnp.bfloat16),
        ],
        in_specs=[
            pl.BlockSpec(memory_space=pl.ANY),
            pl.BlockSpec(memory_space=pl.ANY),
        ],
        out_specs=[
            pl.BlockSpec(memory_space=pl.ANY),
            pl.BlockSpec(memory_space=pl.ANY),
        ],
        scratch_shapes=[
            pltpu.VMEM((M, K), jnp.bfloat16),
            pltpu.VMEM((2, M, CK), jnp.float32),
            pltpu.VMEM((2, K, TN), jnp.float32),
            pltpu.VMEM((2, K, TN), jnp.bfloat16),
            pltpu.VMEM((NSLOT, M, TN), jnp.bfloat16),
            pltpu.VMEM((M, TN), jnp.bfloat16),
            pltpu.VMEM((M, TN), jnp.float32),
            pltpu.SemaphoreType.DMA((2,)),
            pltpu.SemaphoreType.DMA((2,)),
            pltpu.SemaphoreType.DMA((T,)),
            pltpu.SemaphoreType.DMA((T,)),
            pltpu.SemaphoreType.DMA((2,)),
        ],
        compiler_params=pltpu.CompilerParams(
            collective_id=0,
            vmem_limit_bytes=64 * 1024 * 1024,
        ),
    )(A, B)
    return out
